# baseline (speedup 1.0000x reference)
import functools

import jax
import jax.numpy as jnp
from jax import lax
from jax.experimental import pallas as pl
from jax.experimental.pallas import tpu as pltpu
from jax.experimental.pallas import tpu_sc as plsc

EMB = 32
GROUP = 16
NUM_WORKERS = 32


@functools.cache
def _build(n_rows, n_fields):
    rows_per_w = n_rows // NUM_WORKERS
    assert rows_per_w % GROUP == 0
    groups_per_w = rows_per_w // GROUP
    mesh = plsc.VectorSubcoreMesh(core_axis_name="c", subcore_axis_name="s")

    @functools.partial(
        pl.kernel,
        mesh=mesh,
        out_type=jax.ShapeDtypeStruct((n_rows, n_fields, EMB), jnp.float32),
        scratch_types=[
            pltpu.VMEM((rows_per_w, n_fields), jnp.int32),
            pltpu.VMEM((2, GROUP, n_fields, EMB), jnp.float32),
            pltpu.SemaphoreType.DMA((2,)),
            pltpu.SemaphoreType.DMA((2,)),
        ],
        compiler_params=pltpu.CompilerParams(use_tc_tiling_on_sc=False),
    )
    def gather_kernel(idx_hbm, table_hbm, out_hbm, idx_v, rows_v, gsem, osem):
        wid = lax.axis_index("s") * 2 + lax.axis_index("c")
        base_row = wid * rows_per_w
        pltpu.sync_copy(idx_hbm.at[pl.ds(base_row, rows_per_w)], idx_v)

        def fire(t, b):
            for j in range(GROUP):
                pltpu.async_copy(
                    table_hbm.at[idx_v.at[t * GROUP + j]],
                    rows_v.at[b, j],
                    gsem.at[b],
                )

        def drain_gathers(b):
            for j in range(GROUP):
                pltpu.make_async_copy(
                    table_hbm.at[idx_v.at[j]],
                    rows_v.at[b, j],
                    gsem.at[b],
                ).wait()

        def store_out(t, b):
            pltpu.async_copy(
                rows_v.at[b],
                out_hbm.at[pl.ds(base_row + t * GROUP, GROUP)],
                osem.at[b],
            )

        def drain_store(t, b):
            pltpu.make_async_copy(
                rows_v.at[b],
                out_hbm.at[pl.ds(base_row + t * GROUP, GROUP)],
                osem.at[b],
            ).wait()

        fire(0, 0)

        def body(t, carry):
            b = t % 2
            @pl.when(t + 1 < groups_per_w)
            def _():
                @pl.when(t >= 1)
                def _():
                    drain_store(t - 1, 1 - b)

                fire(t + 1, 1 - b)

            drain_gathers(b)
            store_out(t, b)
            return carry

        lax.fori_loop(0, groups_per_w, body, 0)
        drain_store(groups_per_w - 2, groups_per_w % 2)
        drain_store(groups_per_w - 1, (groups_per_w - 1) % 2)

    return gather_kernel


def kernel(indices, table):
    n_rows, n_fields = indices.shape
    return _build(n_rows, n_fields)(indices.astype(jnp.int32), table)
